# baseline (device time: 86534 ns/iter reference)
import jax
import jax.numpy as jnp
from jax import lax
from jax.experimental import pallas as pl
from jax.experimental.pallas import tpu as pltpu

N_DEV = 32
B, SQ, SKV, DH = 2, 256, 256, 64
H_LOC = 4
D_MODEL = 512
WINDOW = 128
N_CHUNK = N_DEV
ROWS_PER_CHUNK = (B * SQ) // N_CHUNK
CHUNKS_PER_B = SQ // ROWS_PER_CHUNK


def kernel(x, Wq, K_ext, V_ext, Wo):
    def body(x_ref, wq_ref, k_hbm, v_hbm, wo_ref, out_ref,
             k_vmem, v_vmem, rs_buf, local_sems, send_sems, rs_sems, ag_sems):
        my_pos = lax.axis_index("i")
        h0 = my_pos * H_LOC

        kv_copies = []
        for b in range(B):
            for src, dst, si in ((k_hbm, k_vmem, 2 * b), (v_hbm, v_vmem, 2 * b + 1)):
                cp = pltpu.make_async_copy(
                    src.at[b, :, pl.ds(h0, H_LOC), :],
                    dst.at[b],
                    local_sems.at[si],
                )
                cp.start()
                kv_copies.append(cp)

        qi = lax.broadcasted_iota(jnp.int32, (SQ, SKV), 0)
        ki = lax.broadcasted_iota(jnp.int32, (SQ, SKV), 1)
        mask = jnp.abs(qi - ki) <= WINDOW
        wq = wq_ref[:].astype(jnp.bfloat16)
        wo = wo_ref[:].astype(jnp.bfloat16)
        q_heads = []
        for b in range(B):
            xb = x_ref[b].astype(jnp.bfloat16)
            q_heads.append(
                jnp.dot(xb, wq, preferred_element_type=jnp.float32))
        for cp in kv_copies:
            cp.wait()
        for b in range(B):
            partial = jnp.zeros((SQ, D_MODEL), jnp.float32)
            for h in range(H_LOC):
                qh = q_heads[b][:, h * DH:(h + 1) * DH].astype(jnp.bfloat16)
                kh = k_vmem[b, :, h, :].astype(jnp.bfloat16)
                s = lax.dot_general(
                    qh, kh, (((1,), (1,)), ((), ())),
                    preferred_element_type=jnp.float32,
                ) * 0.125
                s = jnp.where(mask, s, -1e9)
                m = jnp.max(s, axis=1, keepdims=True)
                w = jnp.exp(s - m)
                w = w / jnp.sum(w, axis=1, keepdims=True)
                ctx = jnp.dot(
                    w.astype(jnp.bfloat16),
                    v_vmem[b, :, h, :].astype(jnp.bfloat16),
                    preferred_element_type=jnp.float32,
                )
                partial = partial + jnp.dot(
                    ctx.astype(jnp.bfloat16), wo[h * DH:(h + 1) * DH, :],
                    preferred_element_type=jnp.float32,
                )
            for cb in range(CHUNKS_PER_B):
                out_ref[b * CHUNKS_PER_B + cb] = (
                    partial[cb * ROWS_PER_CHUNK:(cb + 1) * ROWS_PER_CHUNK, :]
                )

        barrier_sem = pltpu.get_barrier_semaphore()
        for d in range(1, N_DEV):
            pl.semaphore_signal(
                barrier_sem, inc=1,
                device_id=((my_pos + d) % N_DEV,),
                device_id_type=pl.DeviceIdType.MESH,
            )
        pl.semaphore_wait(barrier_sem, N_DEV - 1)

        sends = []

        for d in range(1, N_DEV):
            j = (my_pos + d) % N_DEV
            rdma = pltpu.make_async_remote_copy(
                src_ref=out_ref.at[j],
                dst_ref=rs_buf.at[my_pos],
                send_sem=send_sems.at[d - 1],
                recv_sem=rs_sems.at[my_pos],
                device_id=(j,),
                device_id_type=pl.DeviceIdType.MESH,
            )
            rdma.start()
            sends.append(rdma)
        rs_buf[my_pos] = out_ref[my_pos]

        for d in range(1, N_DEV):
            j = (my_pos + d) % N_DEV
            recv = pltpu.make_async_remote_copy(
                src_ref=rs_buf.at[j],
                dst_ref=rs_buf.at[j],
                send_sem=send_sems.at[d - 1],
                recv_sem=rs_sems.at[j],
                device_id=(j,),
                device_id_type=pl.DeviceIdType.MESH,
            )
            recv.wait_recv()

        out_ref[my_pos] = jnp.sum(rs_buf[:], axis=0)

        for d in range(1, N_DEV):
            j = (my_pos + d) % N_DEV
            rdma = pltpu.make_async_remote_copy(
                src_ref=out_ref.at[my_pos],
                dst_ref=out_ref.at[my_pos],
                send_sem=send_sems.at[N_DEV - 2 + d],
                recv_sem=ag_sems.at[my_pos],
                device_id=(j,),
                device_id_type=pl.DeviceIdType.MESH,
            )
            rdma.start()
            sends.append(rdma)

        for d in range(1, N_DEV):
            j = (my_pos + d) % N_DEV
            recv = pltpu.make_async_remote_copy(
                src_ref=out_ref.at[j],
                dst_ref=out_ref.at[j],
                send_sem=send_sems.at[0],
                recv_sem=ag_sems.at[j],
                device_id=(j,),
                device_id_type=pl.DeviceIdType.MESH,
            )
            recv.wait_recv()

        for rdma in sends:
            rdma.wait_send()

    out = pl.pallas_call(
        body,
        out_shape=jax.ShapeDtypeStruct(
            (N_CHUNK, ROWS_PER_CHUNK, D_MODEL), jnp.float32),
        in_specs=[
            pl.BlockSpec(memory_space=pltpu.VMEM),
            pl.BlockSpec(memory_space=pltpu.VMEM),
            pl.BlockSpec(memory_space=pl.ANY),
            pl.BlockSpec(memory_space=pl.ANY),
            pl.BlockSpec(memory_space=pltpu.VMEM),
        ],
        out_specs=pl.BlockSpec(memory_space=pltpu.VMEM),
        scratch_shapes=[
            pltpu.VMEM((B, SKV, H_LOC, DH), jnp.float32),
            pltpu.VMEM((B, SKV, H_LOC, DH), jnp.float32),
            pltpu.VMEM((N_DEV, ROWS_PER_CHUNK, D_MODEL), jnp.float32),
            pltpu.SemaphoreType.DMA((4,)),
            pltpu.SemaphoreType.DMA((2 * (N_DEV - 1),)),
            pltpu.SemaphoreType.DMA((N_DEV,)),
            pltpu.SemaphoreType.DMA((N_DEV,)),
        ],
        compiler_params=pltpu.CompilerParams(collective_id=0),
    )(x, Wq, K_ext, V_ext, Wo)
    return out.reshape(B, SQ, D_MODEL)


# device time: 67859 ns/iter; 1.2752x vs baseline; 1.2752x over previous
import jax
import jax.numpy as jnp
from jax import lax
from jax.experimental import pallas as pl
from jax.experimental.pallas import tpu as pltpu

N_DEV = 32
B, SQ, SKV, DH = 2, 256, 256, 64
H_LOC = 4
D_MODEL = 512
WINDOW = 128
N_CHUNK = N_DEV
ROWS_PER_CHUNK = (B * SQ) // N_CHUNK
CHUNKS_PER_B = SQ // ROWS_PER_CHUNK


def kernel(x, Wq, K_ext, V_ext, Wo):
    my = lax.axis_index("i")
    K_loc = lax.dynamic_slice_in_dim(
        K_ext.astype(jnp.bfloat16), my * H_LOC, H_LOC, axis=2)
    V_loc = lax.dynamic_slice_in_dim(
        V_ext.astype(jnp.bfloat16), my * H_LOC, H_LOC, axis=2)

    def body(x_ref, wq_ref, k_ref, v_ref, wo_ref, out_ref,
             rs_buf, send_sems, rs_sems, ag_sems):
        my_pos = lax.axis_index("i")

        qi = lax.broadcasted_iota(jnp.int32, (SQ, SKV), 0)
        ki = lax.broadcasted_iota(jnp.int32, (SQ, SKV), 1)
        mask = jnp.abs(qi - ki) <= WINDOW
        wq = wq_ref[:].astype(jnp.bfloat16)
        wo = wo_ref[:].astype(jnp.bfloat16)
        for b in range(B):
            xb = x_ref[b].astype(jnp.bfloat16)
            q_all = jnp.dot(xb, wq, preferred_element_type=jnp.float32)
            partial = jnp.zeros((SQ, D_MODEL), jnp.float32)
            for h in range(H_LOC):
                qh = q_all[:, h * DH:(h + 1) * DH].astype(jnp.bfloat16)
                kh = k_ref[b, :, h, :]
                s = lax.dot_general(
                    qh, kh, (((1,), (1,)), ((), ())),
                    preferred_element_type=jnp.float32,
                ) * 0.125
                s = jnp.where(mask, s, -1e9)
                m = jnp.max(s, axis=1, keepdims=True)
                w = jnp.exp(s - m)
                w = w / jnp.sum(w, axis=1, keepdims=True)
                ctx = jnp.dot(
                    w.astype(jnp.bfloat16), v_ref[b, :, h, :],
                    preferred_element_type=jnp.float32,
                )
                partial = partial + jnp.dot(
                    ctx.astype(jnp.bfloat16), wo[h * DH:(h + 1) * DH, :],
                    preferred_element_type=jnp.float32,
                )
            for cb in range(CHUNKS_PER_B):
                out_ref[b * CHUNKS_PER_B + cb] = (
                    partial[cb * ROWS_PER_CHUNK:(cb + 1) * ROWS_PER_CHUNK, :]
                )

        barrier_sem = pltpu.get_barrier_semaphore()
        for d in range(1, N_DEV):
            pl.semaphore_signal(
                barrier_sem, inc=1,
                device_id=((my_pos + d) % N_DEV,),
                device_id_type=pl.DeviceIdType.MESH,
            )
        pl.semaphore_wait(barrier_sem, N_DEV - 1)

        sends = []

        for d in range(1, N_DEV):
            j = (my_pos + d) % N_DEV
            rdma = pltpu.make_async_remote_copy(
                src_ref=out_ref.at[j],
                dst_ref=rs_buf.at[my_pos],
                send_sem=send_sems.at[d - 1],
                recv_sem=rs_sems.at[my_pos],
                device_id=(j,),
                device_id_type=pl.DeviceIdType.MESH,
            )
            rdma.start()
            sends.append(rdma)
        rs_buf[my_pos] = out_ref[my_pos]

        for d in range(1, N_DEV):
            j = (my_pos + d) % N_DEV
            recv = pltpu.make_async_remote_copy(
                src_ref=rs_buf.at[j],
                dst_ref=rs_buf.at[j],
                send_sem=send_sems.at[d - 1],
                recv_sem=rs_sems.at[j],
                device_id=(j,),
                device_id_type=pl.DeviceIdType.MESH,
            )
            recv.wait_recv()

        out_ref[my_pos] = jnp.sum(rs_buf[:], axis=0)

        for d in range(1, N_DEV):
            j = (my_pos + d) % N_DEV
            rdma = pltpu.make_async_remote_copy(
                src_ref=out_ref.at[my_pos],
                dst_ref=out_ref.at[my_pos],
                send_sem=send_sems.at[N_DEV - 2 + d],
                recv_sem=ag_sems.at[my_pos],
                device_id=(j,),
                device_id_type=pl.DeviceIdType.MESH,
            )
            rdma.start()
            sends.append(rdma)

        for d in range(1, N_DEV):
            j = (my_pos + d) % N_DEV
            recv = pltpu.make_async_remote_copy(
                src_ref=out_ref.at[j],
                dst_ref=out_ref.at[j],
                send_sem=send_sems.at[0],
                recv_sem=ag_sems.at[j],
                device_id=(j,),
                device_id_type=pl.DeviceIdType.MESH,
            )
            recv.wait_recv()

        for rdma in sends:
            rdma.wait_send()

    out = pl.pallas_call(
        body,
        out_shape=jax.ShapeDtypeStruct(
            (N_CHUNK, ROWS_PER_CHUNK, D_MODEL), jnp.float32),
        in_specs=[pl.BlockSpec(memory_space=pltpu.VMEM)] * 5,
        out_specs=pl.BlockSpec(memory_space=pltpu.VMEM),
        scratch_shapes=[
            pltpu.VMEM((N_DEV, ROWS_PER_CHUNK, D_MODEL), jnp.float32),
            pltpu.SemaphoreType.DMA((2 * (N_DEV - 1),)),
            pltpu.SemaphoreType.DMA((N_DEV,)),
            pltpu.SemaphoreType.DMA((N_DEV,)),
        ],
        compiler_params=pltpu.CompilerParams(collective_id=0),
    )(x, Wq, K_loc, V_loc, Wo)
    return out.reshape(B, SQ, D_MODEL)


# device time: 66203 ns/iter; 1.3071x vs baseline; 1.0250x over previous
import jax
import jax.numpy as jnp
from jax import lax
from jax.experimental import pallas as pl
from jax.experimental.pallas import tpu as pltpu

N_DEV = 32
B, SQ, SKV, DH = 2, 256, 256, 64
H_LOC = 4
D_MODEL = 512
WINDOW = 128
N_CHUNK = N_DEV
ROWS_PER_CHUNK = (B * SQ) // N_CHUNK
CHUNKS_PER_B = SQ // ROWS_PER_CHUNK


def kernel(x, Wq, K_ext, V_ext, Wo):
    my = lax.axis_index("i")
    K_loc = lax.dynamic_slice_in_dim(
        K_ext.astype(jnp.bfloat16), my * H_LOC, H_LOC, axis=2
    ).reshape(B, SKV, H_LOC * DH)
    V_loc = lax.dynamic_slice_in_dim(
        V_ext.astype(jnp.bfloat16), my * H_LOC, H_LOC, axis=2
    ).reshape(B, SKV, H_LOC * DH)

    def body(x_ref, wq_ref, k_ref, v_ref, wo_ref, out_ref,
             rs_buf, send_sems, rs_sems, ag_sems):
        my_pos = lax.axis_index("i")

        qi = lax.broadcasted_iota(jnp.int32, (SQ, SKV), 0)
        ki = lax.broadcasted_iota(jnp.int32, (SQ, SKV), 1)
        mask = jnp.abs(qi - ki) <= WINDOW
        wq = wq_ref[:].astype(jnp.bfloat16)
        wo = wo_ref[:].astype(jnp.bfloat16)
        for b in range(B):
            xb = x_ref[b].astype(jnp.bfloat16)
            q_all = jnp.dot(xb, wq, preferred_element_type=jnp.float32)
            partial = jnp.zeros((SQ, D_MODEL), jnp.float32)
            for h in range(H_LOC):
                qh = q_all[:, h * DH:(h + 1) * DH].astype(jnp.bfloat16)
                kh = k_ref[b, :, h * DH:(h + 1) * DH]
                s = lax.dot_general(
                    qh, kh, (((1,), (1,)), ((), ())),
                    preferred_element_type=jnp.float32,
                ) * 0.125
                s = jnp.where(mask, s, -1e9)
                m = jnp.max(s, axis=1, keepdims=True)
                w = jnp.exp(s - m)
                w = w / jnp.sum(w, axis=1, keepdims=True)
                ctx = jnp.dot(
                    w.astype(jnp.bfloat16), v_ref[b, :, h * DH:(h + 1) * DH],
                    preferred_element_type=jnp.float32,
                )
                partial = partial + jnp.dot(
                    ctx.astype(jnp.bfloat16), wo[h * DH:(h + 1) * DH, :],
                    preferred_element_type=jnp.float32,
                )
            for cb in range(CHUNKS_PER_B):
                out_ref[b * CHUNKS_PER_B + cb] = (
                    partial[cb * ROWS_PER_CHUNK:(cb + 1) * ROWS_PER_CHUNK, :]
                )

        barrier_sem = pltpu.get_barrier_semaphore()
        for d in range(1, N_DEV):
            pl.semaphore_signal(
                barrier_sem, inc=1,
                device_id=((my_pos + d) % N_DEV,),
                device_id_type=pl.DeviceIdType.MESH,
            )
        pl.semaphore_wait(barrier_sem, N_DEV - 1)

        sends = []

        for d in range(1, N_DEV):
            j = (my_pos + d) % N_DEV
            rdma = pltpu.make_async_remote_copy(
                src_ref=out_ref.at[j],
                dst_ref=rs_buf.at[my_pos],
                send_sem=send_sems.at[d - 1],
                recv_sem=rs_sems.at[my_pos],
                device_id=(j,),
                device_id_type=pl.DeviceIdType.MESH,
            )
            rdma.start()
            sends.append(rdma)
        rs_buf[my_pos] = out_ref[my_pos]

        for d in range(1, N_DEV):
            j = (my_pos + d) % N_DEV
            recv = pltpu.make_async_remote_copy(
                src_ref=rs_buf.at[j],
                dst_ref=rs_buf.at[j],
                send_sem=send_sems.at[d - 1],
                recv_sem=rs_sems.at[j],
                device_id=(j,),
                device_id_type=pl.DeviceIdType.MESH,
            )
            recv.wait_recv()

        out_ref[my_pos] = jnp.sum(rs_buf[:], axis=0)

        for d in range(1, N_DEV):
            j = (my_pos + d) % N_DEV
            rdma = pltpu.make_async_remote_copy(
                src_ref=out_ref.at[my_pos],
                dst_ref=out_ref.at[my_pos],
                send_sem=send_sems.at[N_DEV - 2 + d],
                recv_sem=ag_sems.at[my_pos],
                device_id=(j,),
                device_id_type=pl.DeviceIdType.MESH,
            )
            rdma.start()
            sends.append(rdma)

        for d in range(1, N_DEV):
            j = (my_pos + d) % N_DEV
            recv = pltpu.make_async_remote_copy(
                src_ref=out_ref.at[j],
                dst_ref=out_ref.at[j],
                send_sem=send_sems.at[0],
                recv_sem=ag_sems.at[j],
                device_id=(j,),
                device_id_type=pl.DeviceIdType.MESH,
            )
            recv.wait_recv()

        for rdma in sends:
            rdma.wait_send()

    out = pl.pallas_call(
        body,
        out_shape=jax.ShapeDtypeStruct(
            (N_CHUNK, ROWS_PER_CHUNK, D_MODEL), jnp.float32),
        in_specs=[pl.BlockSpec(memory_space=pltpu.VMEM)] * 5,
        out_specs=pl.BlockSpec(memory_space=pltpu.VMEM),
        scratch_shapes=[
            pltpu.VMEM((N_DEV, ROWS_PER_CHUNK, D_MODEL), jnp.float32),
            pltpu.SemaphoreType.DMA((2 * (N_DEV - 1),)),
            pltpu.SemaphoreType.DMA((N_DEV,)),
            pltpu.SemaphoreType.DMA((N_DEV,)),
        ],
        compiler_params=pltpu.CompilerParams(collective_id=0),
    )(x, Wq, K_loc, V_loc, Wo)
    return out.reshape(B, SQ, D_MODEL)
